# baseline (speedup 1.0000x reference)
import functools

import jax
import jax.numpy as jnp
from jax import lax
from jax.experimental import pallas as pl
from jax.experimental.pallas import tpu as pltpu
from jax.experimental.pallas import tpu_sc as plsc

_N = 10000
_E = 320000
_IN = 31
_HID = 128
_NCLS = 20

_NC = 2
_NS = 16
_NT = _NC * _NS
_EPT = _E // _NT
_K = 80
_NCHUNK = _EPT // _K
_ACC_ROWS = 10240
_ZROWS = 128
_RPT = 624
_TAIL = _N - _RPT * _NS

_mesh = plsc.VectorSubcoreMesh(core_axis_name="c", subcore_axis_name="s")


def _zero_fill(ref, rows, width):
    per_row = width // 16

    def body(i, _):
        ref[i // per_row, pl.ds((i % per_row) * 16, 16)] = jnp.zeros(
            (16,), jnp.float32)
        return 0

    lax.fori_loop(0, rows * per_row, body, 0)


def _unpack_chunk(packed_v, j, src_c, dst_c):
    for i in range(_K // 16):
        v = packed_v[j, pl.ds(i * 16, 16)]
        if src_c is not None:
            src_c[pl.ds(i * 16, 16)] = jnp.bitwise_and(v, 0xFFFF)
        dst_c[pl.ds(i * 16, 16)] = lax.shift_right_logical(v, 16)


@functools.partial(
    pl.kernel,
    mesh=_mesh,
    out_type=jax.ShapeDtypeStruct((_NC, _N, 16), jnp.float32),
    scratch_types=[
        pltpu.VMEM((_NCHUNK, _K), jnp.int32),
        pltpu.VMEM((_K,), jnp.int32),
        pltpu.VMEM((_K, 16), jnp.float32),
        pltpu.VMEM((_ZROWS, 16), jnp.float32),
        pltpu.VMEM_SHARED((_ACC_ROWS, 16), jnp.float32),
    ],
)
def _deg_kernel(packed_hbm, out_hbm, packed_v, dst_c, ones_v, zero_v, acc_sh):
    c = lax.axis_index("c")
    s = lax.axis_index("s")
    tid = c * _NS + s

    def ofill(i, _):
        ones_v[i] = jnp.ones((16,), jnp.float32)
        return 0

    lax.fori_loop(0, _K, ofill, 0)
    _zero_fill(zero_v, _ZROWS, 16)

    def zinit(j, _):
        pltpu.sync_copy(zero_v, acc_sh.at[pl.ds(s * 640 + j * _ZROWS, _ZROWS)])
        return 0

    lax.fori_loop(0, 640 // _ZROWS, zinit, 0)
    pltpu.sync_copy(packed_hbm.at[tid], packed_v)
    plsc.subcore_barrier()

    def body(j, _):
        _unpack_chunk(packed_v, j, None, dst_c)
        pltpu.sync_copy(ones_v, acc_sh.at[dst_c], add=True)
        return 0

    lax.fori_loop(0, _NCHUNK, body, 0)
    plsc.subcore_barrier()
    pltpu.sync_copy(acc_sh.at[pl.ds(s * _RPT, _RPT)],
                    out_hbm.at[c, pl.ds(s * _RPT, _RPT)])

    @pl.when(s == 0)
    def _():
        pltpu.sync_copy(acc_sh.at[pl.ds(_RPT * _NS, _TAIL)],
                        out_hbm.at[c, pl.ds(_RPT * _NS, _TAIL)])


@functools.partial(
    pl.kernel,
    mesh=_mesh,
    out_type=jax.ShapeDtypeStruct((_NC, _N, _HID), jnp.float32),
    scratch_types=[
        pltpu.VMEM((_NCHUNK, _K), jnp.int32),
        pltpu.VMEM((_K,), jnp.int32),
        pltpu.VMEM((_K,), jnp.int32),
        pltpu.VMEM((_K,), jnp.int32),
        pltpu.VMEM((_K,), jnp.int32),
        pltpu.VMEM((_K, _HID), jnp.float32),
        pltpu.VMEM((_K, _HID), jnp.float32),
        pltpu.VMEM_SHARED((_ACC_ROWS, _HID), jnp.float32),
        pltpu.SemaphoreType.DMA,
        pltpu.SemaphoreType.DMA,
        pltpu.SemaphoreType.DMA,
        pltpu.SemaphoreType.DMA,
    ],
)
def _scatter_kernel(y_hbm, packed_hbm, out_hbm, packed_v,
                    src_c0, dst_c0, src_c1, dst_c1, buf0, buf1,
                    acc_sh, sem0, sem1, ssem0, ssem1):
    c = lax.axis_index("c")
    s = lax.axis_index("s")
    tid = c * _NS + s

    _zero_fill(buf0, _K, _HID)

    def zinit(j, _):
        pltpu.sync_copy(buf0, acc_sh.at[pl.ds(s * 640 + j * _K, _K)])
        return 0

    lax.fori_loop(0, 640 // _K, zinit, 0)
    pltpu.sync_copy(packed_hbm.at[tid], packed_v)
    plsc.subcore_barrier()

    def wait_g(src_c, buf, sem):
        pltpu.make_async_copy(y_hbm.at[src_c], buf, sem).wait()

    def wait_s(dst_c, buf, ssem):
        pltpu.make_async_copy(buf, acc_sh.at[dst_c], ssem).wait()

    _unpack_chunk(packed_v, 0, src_c0, dst_c0)
    pltpu.async_copy(y_hbm.at[src_c0], buf0, sem0)
    _unpack_chunk(packed_v, 1, src_c1, dst_c1)
    pltpu.async_copy(y_hbm.at[src_c1], buf1, sem1)
    wait_g(src_c0, buf0, sem0)
    pltpu.async_copy(buf0, acc_sh.at[dst_c0], ssem0, add=True)

    def body(jj, _):
        j = 2 * jj + 1
        wait_g(src_c1, buf1, sem1)
        pltpu.async_copy(buf1, acc_sh.at[dst_c1], ssem1, add=True)
        wait_s(dst_c0, buf0, ssem0)
        _unpack_chunk(packed_v, j + 1, src_c0, dst_c0)
        pltpu.async_copy(y_hbm.at[src_c0], buf0, sem0)
        wait_g(src_c0, buf0, sem0)
        pltpu.async_copy(buf0, acc_sh.at[dst_c0], ssem0, add=True)
        wait_s(dst_c1, buf1, ssem1)
        _unpack_chunk(packed_v, j + 2, src_c1, dst_c1)
        pltpu.async_copy(y_hbm.at[src_c1], buf1, sem1)
        return 0

    lax.fori_loop(0, (_NCHUNK - 3) // 2, body, 0)
    wait_g(src_c1, buf1, sem1)
    pltpu.async_copy(buf1, acc_sh.at[dst_c1], ssem1, add=True)
    wait_s(dst_c0, buf0, ssem0)
    _unpack_chunk(packed_v, _NCHUNK - 1, src_c0, dst_c0)
    pltpu.async_copy(y_hbm.at[src_c0], buf0, sem0)
    wait_g(src_c0, buf0, sem0)
    pltpu.async_copy(buf0, acc_sh.at[dst_c0], ssem0, add=True)
    wait_s(dst_c1, buf1, ssem1)
    wait_s(dst_c0, buf0, ssem0)
    plsc.subcore_barrier()
    pltpu.sync_copy(acc_sh.at[pl.ds(s * _RPT, _RPT)],
                    out_hbm.at[c, pl.ds(s * _RPT, _RPT)])

    @pl.when(s == 0)
    def _():
        pltpu.sync_copy(acc_sh.at[pl.ds(_RPT * _NS, _TAIL)],
                        out_hbm.at[c, pl.ds(_RPT * _NS, _TAIL)])


_BLK = 1000


def _dinv_block(degp_ref):
    deg = 1.0 + degp_ref[0, :, 0:1] + degp_ref[1, :, 0:1]
    return lax.rsqrt(deg)


def _enc_body(x_ref, we_ref, be_ref, w1_ref, degp_ref, y1_ref):
    dinv = _dinv_block(degp_ref)
    h0 = jnp.maximum(
        jnp.dot(x_ref[...], we_ref[...], preferred_element_type=jnp.float32)
        + be_ref[...], 0.0)
    y1_ref[...] = dinv * jnp.dot(
        h0, w1_ref[...], preferred_element_type=jnp.float32)


def _mid_body(acc_ref, y_ref, degp_ref, b_ref, w_ref, out_ref):
    dinv = _dinv_block(degp_ref)
    h = jnp.maximum(
        dinv * (acc_ref[0] + acc_ref[1] + y_ref[...]) + b_ref[...], 0.0)
    out_ref[...] = dinv * jnp.dot(
        h, w_ref[...], preferred_element_type=jnp.float32)


def _out_body(acc_ref, y_ref, degp_ref, b_ref, wc_ref, bc_ref, out_ref):
    dinv = _dinv_block(degp_ref)
    h = jnp.maximum(
        dinv * (acc_ref[0] + acc_ref[1] + y_ref[...]) + b_ref[...], 0.0)
    out_ref[...] = jnp.dot(
        h, wc_ref[...], preferred_element_type=jnp.float32) + bc_ref[...]


_full = lambda *dims: pl.BlockSpec(dims, lambda i: tuple(0 for _ in dims))
_rows = lambda *dims: pl.BlockSpec(dims, lambda i: (i,) + tuple(
    0 for _ in dims[1:]))
_acc_spec = pl.BlockSpec((_NC, _BLK, _HID), lambda i: (0, i, 0))
_degp_spec = pl.BlockSpec((_NC, _BLK, 16), lambda i: (0, i, 0))

_enc_call = pl.pallas_call(
    _enc_body,
    grid=(_N // _BLK,),
    in_specs=[_rows(_BLK, 32), _full(32, _HID), _full(1, _HID),
              _full(_HID, _HID), _degp_spec],
    out_specs=_rows(_BLK, _HID),
    out_shape=jax.ShapeDtypeStruct((_N, _HID), jnp.float32),
)

_mid_call = pl.pallas_call(
    _mid_body,
    grid=(_N // _BLK,),
    in_specs=[_acc_spec, _rows(_BLK, _HID), _degp_spec, _full(1, _HID),
              _full(_HID, _HID)],
    out_specs=_rows(_BLK, _HID),
    out_shape=jax.ShapeDtypeStruct((_N, _HID), jnp.float32),
)

_out_call = pl.pallas_call(
    _out_body,
    grid=(_N // _BLK,),
    in_specs=[_acc_spec, _rows(_BLK, _HID), _degp_spec, _full(1, _HID),
              _full(_HID, _NCLS), _full(1, _NCLS)],
    out_specs=_rows(_BLK, _NCLS),
    out_shape=jax.ShapeDtypeStruct((_N, _NCLS), jnp.float32),
)


def kernel(x, edge_index, W_enc, b_enc, W1, b1, W2, b2, W_cls, b_cls):
    packed = jnp.bitwise_or(
        jnp.left_shift(edge_index[1], 16), edge_index[0]
    ).reshape(_NT, _NCHUNK, _K)
    xp = jnp.pad(x, ((0, 0), (0, 1)))
    Wep = jnp.pad(W_enc, ((0, 1), (0, 0)))

    degp = _deg_kernel(packed)
    y1 = _enc_call(xp, Wep, b_enc.reshape(1, _HID), W1, degp)
    acc1 = _scatter_kernel(y1, packed)
    y2 = _mid_call(acc1, y1, degp, b1.reshape(1, _HID), W2)
    acc2 = _scatter_kernel(y2, packed)
    return _out_call(acc2, y2, degp, b2.reshape(1, _HID), W_cls,
                     b_cls.reshape(1, _NCLS))


# speedup vs baseline: 1.2331x; 1.2331x over previous
import functools

import jax
import jax.numpy as jnp
from jax import lax
from jax.experimental import pallas as pl
from jax.experimental.pallas import tpu as pltpu
from jax.experimental.pallas import tpu_sc as plsc

_N = 10000
_E = 320000
_IN = 31
_HID = 128
_NCLS = 20

_NC = 2
_NS = 16
_NT = _NC * _NS
_EPT = _E // _NT
_K = 80
_NCHUNK = _EPT // _K
_ACC_ROWS = 10240
_ZROWS = 128
_RPT = 624
_TAIL = _N - _RPT * _NS

_mesh = plsc.VectorSubcoreMesh(core_axis_name="c", subcore_axis_name="s")


def _zero_fill(ref, rows, width):
    per_row = width // 16

    def body(i, _):
        ref[i // per_row, pl.ds((i % per_row) * 16, 16)] = jnp.zeros(
            (16,), jnp.float32)
        return 0

    lax.fori_loop(0, rows * per_row, body, 0)


def _unpack_chunk(packed_v, j, src_c, dst_c):
    for i in range(_K // 16):
        v = packed_v[j, pl.ds(i * 16, 16)]
        if src_c is not None:
            src_c[pl.ds(i * 16, 16)] = jnp.bitwise_and(v, 0xFFFF)
        dst_c[pl.ds(i * 16, 16)] = lax.shift_right_logical(v, 16)


@functools.partial(
    pl.kernel,
    mesh=_mesh,
    out_type=jax.ShapeDtypeStruct((_NC, _N, 16), jnp.float32),
    scratch_types=[
        pltpu.VMEM((_NCHUNK, _K), jnp.int32),
        pltpu.VMEM((_K,), jnp.int32),
        pltpu.VMEM((_K, 16), jnp.float32),
        pltpu.VMEM((_ZROWS, 16), jnp.float32),
        pltpu.VMEM_SHARED((_ACC_ROWS, 16), jnp.float32),
    ],
)
def _deg_kernel(packed_hbm, out_hbm, packed_v, dst_c, ones_v, zero_v, acc_sh):
    c = lax.axis_index("c")
    s = lax.axis_index("s")
    tid = c * _NS + s

    def ofill(i, _):
        ones_v[i] = jnp.ones((16,), jnp.float32)
        return 0

    lax.fori_loop(0, _K, ofill, 0)
    _zero_fill(zero_v, _ZROWS, 16)

    def zinit(j, _):
        pltpu.sync_copy(zero_v, acc_sh.at[pl.ds(s * 640 + j * _ZROWS, _ZROWS)])
        return 0

    lax.fori_loop(0, 640 // _ZROWS, zinit, 0)
    pltpu.sync_copy(packed_hbm.at[tid], packed_v)
    plsc.subcore_barrier()

    def body(j, _):
        _unpack_chunk(packed_v, j, None, dst_c)
        pltpu.sync_copy(ones_v, acc_sh.at[dst_c], add=True)
        return 0

    lax.fori_loop(0, _NCHUNK, body, 0)
    plsc.subcore_barrier()
    pltpu.sync_copy(acc_sh.at[pl.ds(s * _RPT, _RPT)],
                    out_hbm.at[c, pl.ds(s * _RPT, _RPT)])

    @pl.when(s == 0)
    def _():
        pltpu.sync_copy(acc_sh.at[pl.ds(_RPT * _NS, _TAIL)],
                        out_hbm.at[c, pl.ds(_RPT * _NS, _TAIL)])


@functools.partial(
    pl.kernel,
    mesh=_mesh,
    out_type=jax.ShapeDtypeStruct((_NC, _N, _HID), jnp.float32),
    scratch_types=[
        pltpu.VMEM((_NCHUNK, _K), jnp.int32),
        pltpu.VMEM((_K,), jnp.int32),
        pltpu.VMEM((_K,), jnp.int32),
        pltpu.VMEM((_K,), jnp.int32),
        pltpu.VMEM((_K,), jnp.int32),
        pltpu.VMEM((_K, _HID), jnp.float32),
        pltpu.VMEM((_K, _HID), jnp.float32),
        pltpu.VMEM_SHARED((_ACC_ROWS, _HID), jnp.float32),
        pltpu.SemaphoreType.DMA,
        pltpu.SemaphoreType.DMA,
    ],
)
def _scatter_kernel(y_hbm, packed_hbm, out_hbm, packed_v,
                    src_c0, dst_c0, src_c1, dst_c1, buf0, buf1,
                    acc_sh, sem0, sem1):
    c = lax.axis_index("c")
    s = lax.axis_index("s")
    tid = c * _NS + s

    _zero_fill(buf0, _K, _HID)

    def zinit(j, _):
        pltpu.sync_copy(buf0, acc_sh.at[pl.ds(s * 640 + j * _K, _K)])
        return 0

    lax.fori_loop(0, 640 // _K, zinit, 0)
    pltpu.sync_copy(packed_hbm.at[tid], packed_v)
    plsc.subcore_barrier()

    _unpack_chunk(packed_v, 0, src_c0, dst_c0)
    pltpu.async_copy(y_hbm.at[src_c0], buf0, sem0)

    def body(jj, _):
        j0 = 2 * jj
        _unpack_chunk(packed_v, j0 + 1, src_c1, dst_c1)
        pltpu.async_copy(y_hbm.at[src_c1], buf1, sem1)
        pltpu.make_async_copy(y_hbm.at[src_c0], buf0, sem0).wait()
        pltpu.sync_copy(buf0, acc_sh.at[dst_c0], add=True)
        _unpack_chunk(packed_v, j0 + 2, src_c0, dst_c0)
        pltpu.async_copy(y_hbm.at[src_c0], buf0, sem0)
        pltpu.make_async_copy(y_hbm.at[src_c1], buf1, sem1).wait()
        pltpu.sync_copy(buf1, acc_sh.at[dst_c1], add=True)
        return 0

    lax.fori_loop(0, _NCHUNK // 2, body, 0)
    pltpu.make_async_copy(y_hbm.at[src_c0], buf0, sem0).wait()
    pltpu.sync_copy(buf0, acc_sh.at[dst_c0], add=True)
    plsc.subcore_barrier()
    pltpu.sync_copy(acc_sh.at[pl.ds(s * _RPT, _RPT)],
                    out_hbm.at[c, pl.ds(s * _RPT, _RPT)])

    @pl.when(s == 0)
    def _():
        pltpu.sync_copy(acc_sh.at[pl.ds(_RPT * _NS, _TAIL)],
                        out_hbm.at[c, pl.ds(_RPT * _NS, _TAIL)])


_BLK = 1000


def _dinv_block(degp_ref):
    deg = 1.0 + degp_ref[0, :, 0:1] + degp_ref[1, :, 0:1]
    return lax.rsqrt(deg)


def _enc_body(x_ref, we_ref, be_ref, w1_ref, degp_ref, y1_ref):
    dinv = _dinv_block(degp_ref)
    h0 = jnp.maximum(
        jnp.dot(x_ref[...], we_ref[...], preferred_element_type=jnp.float32)
        + be_ref[...], 0.0)
    y1_ref[...] = dinv * jnp.dot(
        h0, w1_ref[...], preferred_element_type=jnp.float32)


def _mid_body(acc_ref, y_ref, degp_ref, b_ref, w_ref, out_ref):
    dinv = _dinv_block(degp_ref)
    h = jnp.maximum(
        dinv * (acc_ref[0] + acc_ref[1] + y_ref[...]) + b_ref[...], 0.0)
    out_ref[...] = dinv * jnp.dot(
        h, w_ref[...], preferred_element_type=jnp.float32)


def _out_body(acc_ref, y_ref, degp_ref, b_ref, wc_ref, bc_ref, out_ref):
    dinv = _dinv_block(degp_ref)
    h = jnp.maximum(
        dinv * (acc_ref[0] + acc_ref[1] + y_ref[...]) + b_ref[...], 0.0)
    out_ref[...] = jnp.dot(
        h, wc_ref[...], preferred_element_type=jnp.float32) + bc_ref[...]


_full = lambda *dims: pl.BlockSpec(dims, lambda i: tuple(0 for _ in dims))
_rows = lambda *dims: pl.BlockSpec(dims, lambda i: (i,) + tuple(
    0 for _ in dims[1:]))
_acc_spec = pl.BlockSpec((_NC, _BLK, _HID), lambda i: (0, i, 0))
_degp_spec = pl.BlockSpec((_NC, _BLK, 16), lambda i: (0, i, 0))

_enc_call = pl.pallas_call(
    _enc_body,
    grid=(_N // _BLK,),
    in_specs=[_rows(_BLK, _IN), _full(_IN, _HID), _full(1, _HID),
              _full(_HID, _HID), _degp_spec],
    out_specs=_rows(_BLK, _HID),
    out_shape=jax.ShapeDtypeStruct((_N, _HID), jnp.float32),
)

_mid_call = pl.pallas_call(
    _mid_body,
    grid=(_N // _BLK,),
    in_specs=[_acc_spec, _rows(_BLK, _HID), _degp_spec, _full(1, _HID),
              _full(_HID, _HID)],
    out_specs=_rows(_BLK, _HID),
    out_shape=jax.ShapeDtypeStruct((_N, _HID), jnp.float32),
)

_out_call = pl.pallas_call(
    _out_body,
    grid=(_N // _BLK,),
    in_specs=[_acc_spec, _rows(_BLK, _HID), _degp_spec, _full(1, _HID),
              _full(_HID, _NCLS), _full(1, _NCLS)],
    out_specs=_rows(_BLK, _NCLS),
    out_shape=jax.ShapeDtypeStruct((_N, _NCLS), jnp.float32),
)


def kernel(x, edge_index, W_enc, b_enc, W1, b1, W2, b2, W_cls, b_cls):
    packed = jnp.bitwise_or(
        jnp.left_shift(edge_index[1], 16), edge_index[0]
    ).reshape(_NT, _NCHUNK, _K)
    degp = _deg_kernel(packed)
    y1 = _enc_call(x, W_enc, b_enc.reshape(1, _HID), W1, degp)
    acc1 = _scatter_kernel(y1, packed)
    y2 = _mid_call(acc1, y1, degp, b1.reshape(1, _HID), W2)
    acc2 = _scatter_kernel(y2, packed)
    return _out_call(acc2, y2, degp, b2.reshape(1, _HID), W_cls,
                     b_cls.reshape(1, _NCLS))
